# Initial kernel scaffold; baseline (speedup 1.0000x reference)
import jax
import jax.numpy as jnp
from jax.experimental import pallas as pl


def kernel(h_inputs, objectives, edge_index, W1, al1, ar1, b1, W2, al2, ar2, b2):
    raise NotImplementedError("write your pallas kernel here")


# speedup vs baseline: 37.1743x; 37.1743x over previous
import functools

import jax
import jax.numpy as jnp
from jax import lax
from jax.experimental import pallas as pl
from jax.experimental.pallas import tpu as pltpu
from jax.experimental.pallas import tpu_sc as plsc

N = 10000
E = 320000
D = 128

NT = 32
EPT = E // NT
CH = 80
NCK = EPT // CH
G = CH // 16
NPT = N // 16

_INTERPRET = False


def _tc_first(x, W, alr):
    BLK = 1280

    def body(x_ref, w_ref, alr_ref, feat_ref, ee_ref):
        feat = jnp.dot(x_ref[...], w_ref[...], preferred_element_type=jnp.float32)
        feat_ref[...] = feat
        ee_ref[...] = jnp.dot(feat, alr_ref[...], preferred_element_type=jnp.float32)

    return pl.pallas_call(
        body,
        grid=(pl.cdiv(N, BLK),),
        in_specs=[
            pl.BlockSpec((BLK, D), lambda i: (i, 0)),
            pl.BlockSpec((D, D), lambda i: (0, 0)),
            pl.BlockSpec((D, 2), lambda i: (0, 0)),
        ],
        out_specs=[
            pl.BlockSpec((BLK, D), lambda i: (i, 0)),
            pl.BlockSpec((BLK, 2), lambda i: (i, 0)),
        ],
        out_shape=[
            jax.ShapeDtypeStruct((N, D), jnp.float32),
            jax.ShapeDtypeStruct((N, 2), jnp.float32),
        ],
        interpret=_INTERPRET,
    )(x, W, alr)


def _combine(part_ref, den_ref, b_ref):
    dsum = jnp.sum(den_ref[...], axis=0)
    return (part_ref[0] + part_ref[1]) / (dsum[:, None] + 1e-9) + b_ref[...]


def _tc_mid(part, den, b, W, alr):
    BLK = 1280

    def body(part_ref, den_ref, b_ref, w_ref, alr_ref, feat_ref, ee_ref):
        h = _combine(part_ref, den_ref, b_ref)
        feat = jnp.dot(h, w_ref[...], preferred_element_type=jnp.float32)
        feat_ref[...] = feat
        ee_ref[...] = jnp.dot(feat, alr_ref[...], preferred_element_type=jnp.float32)

    return pl.pallas_call(
        body,
        grid=(pl.cdiv(N, BLK),),
        in_specs=[
            pl.BlockSpec((2, BLK, D), lambda i: (0, i, 0)),
            pl.BlockSpec((NT, BLK), lambda i: (0, i)),
            pl.BlockSpec((1, D), lambda i: (0, 0)),
            pl.BlockSpec((D, D), lambda i: (0, 0)),
            pl.BlockSpec((D, 2), lambda i: (0, 0)),
        ],
        out_specs=[
            pl.BlockSpec((BLK, D), lambda i: (i, 0)),
            pl.BlockSpec((BLK, 2), lambda i: (i, 0)),
        ],
        out_shape=[
            jax.ShapeDtypeStruct((N, D), jnp.float32),
            jax.ShapeDtypeStruct((N, 2), jnp.float32),
        ],
        interpret=_INTERPRET,
    )(part, den, b, W, alr)


def _tc_final(part, den, b):
    BLK = 1280

    def body(part_ref, den_ref, b_ref, h_ref):
        h_ref[...] = _combine(part_ref, den_ref, b_ref)

    return pl.pallas_call(
        body,
        grid=(pl.cdiv(N, BLK),),
        in_specs=[
            pl.BlockSpec((2, BLK, D), lambda i: (0, i, 0)),
            pl.BlockSpec((NT, BLK), lambda i: (0, i)),
            pl.BlockSpec((1, D), lambda i: (0, 0)),
        ],
        out_specs=pl.BlockSpec((BLK, D), lambda i: (i, 0)),
        out_shape=jax.ShapeDtypeStruct((N, D), jnp.float32),
        interpret=_INTERPRET,
    )(part, den, b)


def _sc_edge(feat, el_h_in, er_h_in, src_flat, dst_flat):
    mesh = plsc.VectorSubcoreMesh(core_axis_name="c", subcore_axis_name="s")

    @functools.partial(
        pl.kernel,
        out_type=[
            jax.ShapeDtypeStruct((2, N, D), jnp.float32),
            jax.ShapeDtypeStruct((NT * N,), jnp.float32),
        ],
        mesh=mesh,
        compiler_params=pltpu.CompilerParams(needs_layout_passes=False),
        scratch_types=[
            pltpu.VMEM((N,), jnp.float32),
            pltpu.VMEM((CH,), jnp.float32),
            pltpu.VMEM((2 * CH, D), jnp.float32),
            pltpu.VMEM((3, CH), jnp.int32),
            pltpu.VMEM((3, CH), jnp.int32),
            pltpu.VMEM((2, CH), jnp.float32),
            pltpu.VMEM((2, CH), jnp.float32),
            pltpu.VMEM_SHARED((N, D), jnp.float32),
            pltpu.SemaphoreType.DMA,
            pltpu.SemaphoreType.DMA,
            pltpu.SemaphoreType.DMA,
        ],
        interpret=_INTERPRET,
    )
    def k(feat_h, el_h, er_h, src_h, dst_h, part_o, den_o,
          den_v, p_v, rows_v, sidx_v, didx_v, elv, erv, acc_sh,
          isem, esem, gsem):
        cid = lax.axis_index("c")
        sid = lax.axis_index("s")
        wid = cid * 16 + sid
        base = wid * EPT

        z16 = jnp.zeros((16,), jnp.float32)

        def zden(i, carry):
            den_v[pl.ds(i * 16, 16)] = z16
            return carry
        lax.fori_loop(0, N // 16, zden, 0)

        def zblk(i, carry):
            for j in range(D // 16):
                rows_v[i, pl.ds(j * 16, 16)] = z16
            return carry
        lax.fori_loop(0, 125, zblk, 0)
        for t in range(NPT // 125):
            pltpu.sync_copy(rows_v.at[pl.ds(0, 125)],
                            acc_sh.at[pl.ds(sid * NPT + t * 125, 125)])
        plsc.subcore_barrier()

        def start_idx(c, slot):
            pltpu.make_async_copy(
                src_h.at[pl.ds(base + c * CH, CH)], sidx_v.at[slot], isem).start()
            pltpu.make_async_copy(
                dst_h.at[pl.ds(base + c * CH, CH)], didx_v.at[slot], isem).start()

        def wait_idx():
            pltpu.make_async_copy(
                src_h.at[pl.ds(base, CH)], sidx_v.at[0], isem).wait()
            pltpu.make_async_copy(
                dst_h.at[pl.ds(base, CH)], didx_v.at[0], isem).wait()

        def start_gathers(b2, b3):
            pltpu.make_async_copy(
                el_h.at[sidx_v.at[b3]], elv.at[b2], esem).start()
            pltpu.make_async_copy(
                er_h.at[didx_v.at[b3]], erv.at[b2], esem).start()
            pltpu.make_async_copy(
                feat_h.at[sidx_v.at[b3]],
                rows_v.at[pl.ds(b2 * CH, CH)], gsem).start()

        start_idx(0, 0)
        start_idx(1, 1)
        wait_idx()
        start_gathers(0, 0)

        def chunk(c, carry):
            b2 = lax.rem(c, 2)
            b3 = lax.rem(c, 3)

            @pl.when(c + 1 < NCK)
            def _():
                wait_idx()
                start_gathers(1 - b2, lax.rem(c + 1, 3))

            @pl.when(c + 2 < NCK)
            def _():
                start_idx(c + 2, lax.rem(c + 2, 3))

            pltpu.make_async_copy(
                el_h.at[sidx_v.at[b3]], elv.at[b2], esem).wait()
            pltpu.make_async_copy(
                er_h.at[didx_v.at[b3]], erv.at[b2], esem).wait()

            for g in range(G):
                sl = pl.ds(g * 16, 16)
                d16 = didx_v[b3, sl]
                z = elv[b2, sl] + erv[b2, sl]
                p16 = jnp.exp(jnp.where(z >= 0, z, z * 0.2))
                plsc.addupdate_scatter(den_v, [d16], p16)
                p_v[sl] = p16

            pltpu.make_async_copy(
                feat_h.at[sidx_v.at[b3]],
                rows_v.at[pl.ds(b2 * CH, CH)], gsem).wait()

            def scale(r, carry2):
                pr = plsc.load_gather(p_v, [jnp.zeros((16,), jnp.int32) + r])
                for j in range(D // 16):
                    sl = pl.ds(j * 16, 16)
                    rows_v[b2 * CH + r, sl] = rows_v[b2 * CH + r, sl] * pr
                return carry2
            lax.fori_loop(0, CH, scale, 0)

            pltpu.sync_copy(rows_v.at[pl.ds(b2 * CH, CH)],
                            acc_sh.at[didx_v.at[b3]], add=True)
            return carry
        lax.fori_loop(0, NCK, chunk, 0)

        plsc.subcore_barrier()

        @pl.when(sid < 15)
        def _():
            pltpu.sync_copy(acc_sh.at[pl.ds(sid * 624, 624)],
                            part_o.at[cid, pl.ds(sid * 624, 624)])

        @pl.when(sid == 15)
        def _():
            pltpu.sync_copy(acc_sh.at[pl.ds(15 * 624, N - 15 * 624)],
                            part_o.at[cid, pl.ds(15 * 624, N - 15 * 624)])

        pltpu.sync_copy(den_v, den_o.at[pl.ds(wid * N, N)])

    return k(feat, el_h_in, er_h_in, src_flat, dst_flat)


def kernel(h_inputs, objectives, edge_index, W1, al1, ar1, b1, W2, al2, ar2, b2):
    x = jnp.concatenate([h_inputs, objectives], axis=1)
    alr1 = jnp.stack([al1.reshape(-1), ar1.reshape(-1)], axis=1)
    alr2 = jnp.stack([al2.reshape(-1), ar2.reshape(-1)], axis=1)
    src_flat = edge_index[0]
    dst_flat = edge_index[1]

    feat1, ee1 = _tc_first(x, W1, alr1)
    part1, den1 = _sc_edge(feat1, ee1[:, 0], ee1[:, 1], src_flat, dst_flat)
    feat2, ee2 = _tc_mid(part1, den1.reshape(NT, N), b1.reshape(1, D), W2, alr2)
    part2, den2 = _sc_edge(feat2, ee2[:, 0], ee2[:, 1], src_flat, dst_flat)
    return _tc_final(part2, den2.reshape(NT, N), b2.reshape(1, D))


# speedup vs baseline: 41.1754x; 1.1076x over previous
import functools

import jax
import jax.numpy as jnp
from jax import lax
from jax.experimental import pallas as pl
from jax.experimental.pallas import tpu as pltpu
from jax.experimental.pallas import tpu_sc as plsc

N = 10000
E = 320000
D = 128

NT = 32
EPT = E // NT
CH = 80
NCK = EPT // CH
G = CH // 16
NPT = N // 16

_INTERPRET = False


def _tc_first(x, W, alr):
    BLK = 1280

    def body(x_ref, w_ref, alr_ref, feat_ref, ee_ref):
        feat = jnp.dot(x_ref[...], w_ref[...], preferred_element_type=jnp.float32)
        feat_ref[...] = feat
        ee_ref[...] = jnp.dot(feat, alr_ref[...], preferred_element_type=jnp.float32)

    return pl.pallas_call(
        body,
        grid=(pl.cdiv(N, BLK),),
        in_specs=[
            pl.BlockSpec((BLK, D), lambda i: (i, 0)),
            pl.BlockSpec((D, D), lambda i: (0, 0)),
            pl.BlockSpec((D, 2), lambda i: (0, 0)),
        ],
        out_specs=[
            pl.BlockSpec((BLK, D), lambda i: (i, 0)),
            pl.BlockSpec((BLK, 2), lambda i: (i, 0)),
        ],
        out_shape=[
            jax.ShapeDtypeStruct((N, D), jnp.float32),
            jax.ShapeDtypeStruct((N, 2), jnp.float32),
        ],
        interpret=_INTERPRET,
    )(x, W, alr)


def _combine(part_ref, den_ref, b_ref):
    dsum = jnp.sum(den_ref[...], axis=0)
    return (part_ref[0] + part_ref[1]) / (dsum[:, None] + 1e-9) + b_ref[...]


def _tc_mid(part, den, b, W, alr):
    BLK = 1280

    def body(part_ref, den_ref, b_ref, w_ref, alr_ref, feat_ref, ee_ref):
        h = _combine(part_ref, den_ref, b_ref)
        feat = jnp.dot(h, w_ref[...], preferred_element_type=jnp.float32)
        feat_ref[...] = feat
        ee_ref[...] = jnp.dot(feat, alr_ref[...], preferred_element_type=jnp.float32)

    return pl.pallas_call(
        body,
        grid=(pl.cdiv(N, BLK),),
        in_specs=[
            pl.BlockSpec((2, BLK, D), lambda i: (0, i, 0)),
            pl.BlockSpec((NT, BLK), lambda i: (0, i)),
            pl.BlockSpec((1, D), lambda i: (0, 0)),
            pl.BlockSpec((D, D), lambda i: (0, 0)),
            pl.BlockSpec((D, 2), lambda i: (0, 0)),
        ],
        out_specs=[
            pl.BlockSpec((BLK, D), lambda i: (i, 0)),
            pl.BlockSpec((BLK, 2), lambda i: (i, 0)),
        ],
        out_shape=[
            jax.ShapeDtypeStruct((N, D), jnp.float32),
            jax.ShapeDtypeStruct((N, 2), jnp.float32),
        ],
        interpret=_INTERPRET,
    )(part, den, b, W, alr)


def _tc_final(part, den, b):
    BLK = 1280

    def body(part_ref, den_ref, b_ref, h_ref):
        h_ref[...] = _combine(part_ref, den_ref, b_ref)

    return pl.pallas_call(
        body,
        grid=(pl.cdiv(N, BLK),),
        in_specs=[
            pl.BlockSpec((2, BLK, D), lambda i: (0, i, 0)),
            pl.BlockSpec((NT, BLK), lambda i: (0, i)),
            pl.BlockSpec((1, D), lambda i: (0, 0)),
        ],
        out_specs=pl.BlockSpec((BLK, D), lambda i: (i, 0)),
        out_shape=jax.ShapeDtypeStruct((N, D), jnp.float32),
        interpret=_INTERPRET,
    )(part, den, b)


def _sc_edge(feat, el_h_in, er_h_in, src_flat, dst_flat):
    mesh = plsc.VectorSubcoreMesh(core_axis_name="c", subcore_axis_name="s")

    @functools.partial(
        pl.kernel,
        out_type=[
            jax.ShapeDtypeStruct((2, N, D), jnp.float32),
            jax.ShapeDtypeStruct((NT * N,), jnp.float32),
        ],
        mesh=mesh,
        compiler_params=pltpu.CompilerParams(needs_layout_passes=False),
        scratch_types=[
            pltpu.VMEM((N,), jnp.float32),
            pltpu.VMEM((CH,), jnp.float32),
            pltpu.VMEM((2 * CH, D), jnp.float32),
            pltpu.VMEM((3, CH), jnp.int32),
            pltpu.VMEM((3, CH), jnp.int32),
            pltpu.VMEM((2, CH), jnp.float32),
            pltpu.VMEM((2, CH), jnp.float32),
            pltpu.VMEM((3, G, 16), jnp.int32),
            pltpu.VMEM_SHARED((N, D), jnp.float32),
            pltpu.SemaphoreType.DMA,
            pltpu.SemaphoreType.DMA,
            pltpu.SemaphoreType.DMA,
            pltpu.SemaphoreType.DMA,
        ],
        interpret=_INTERPRET,
    )
    def k(feat_h, el_h, er_h, src_h, dst_h, part_o, den_o,
          den_v, p_v, rows_v, sidx_v, didx_v, elv, erv, didx_q, acc_sh,
          isem, esem, gsem, ssem):
        cid = lax.axis_index("c")
        sid = lax.axis_index("s")
        wid = cid * 16 + sid
        base = wid * EPT

        z16 = jnp.zeros((16,), jnp.float32)

        def zden(i, carry):
            den_v[pl.ds(i * 16, 16)] = z16
            return carry
        lax.fori_loop(0, N // 16, zden, 0)

        def zblk(i, carry):
            for j in range(D // 16):
                rows_v[i, pl.ds(j * 16, 16)] = z16
            return carry
        lax.fori_loop(0, 125, zblk, 0)
        for t in range(NPT // 125):
            pltpu.sync_copy(rows_v.at[pl.ds(0, 125)],
                            acc_sh.at[pl.ds(sid * NPT + t * 125, 125)])
        plsc.subcore_barrier()

        def start_idx(c, slot):
            pltpu.make_async_copy(
                src_h.at[pl.ds(base + c * CH, CH)], sidx_v.at[slot], isem).start()
            pltpu.make_async_copy(
                dst_h.at[pl.ds(base + c * CH, CH)], didx_v.at[slot], isem).start()
            for g in range(G):
                pltpu.make_async_copy(
                    dst_h.at[pl.ds(base + c * CH + g * 16, 16)],
                    didx_q.at[slot, g], isem).start()

        def wait_idx():
            pltpu.make_async_copy(
                src_h.at[pl.ds(base, CH)], sidx_v.at[0], isem).wait()
            pltpu.make_async_copy(
                dst_h.at[pl.ds(base, CH)], didx_v.at[0], isem).wait()
            for g in range(G):
                pltpu.make_async_copy(
                    dst_h.at[pl.ds(base, 16)], didx_q.at[0, g], isem).wait()

        def start_gathers(b2, b3):
            pltpu.make_async_copy(
                el_h.at[sidx_v.at[b3]], elv.at[b2], esem).start()
            pltpu.make_async_copy(
                er_h.at[didx_v.at[b3]], erv.at[b2], esem).start()
            pltpu.make_async_copy(
                feat_h.at[sidx_v.at[b3]],
                rows_v.at[pl.ds(b2 * CH, CH)], gsem).start()

        start_idx(0, 0)
        start_idx(1, 1)
        wait_idx()
        start_gathers(0, 0)

        def wait_qscatters(b2, b3):
            for g in range(G):
                pltpu.make_async_copy(
                    rows_v.at[pl.ds(b2 * CH + g * 16, 16)],
                    acc_sh.at[didx_q.at[b3, g]], ssem).wait()

        def chunk(c, carry):
            b2 = lax.rem(c, 2)
            b3 = lax.rem(c, 3)

            pltpu.make_async_copy(
                el_h.at[sidx_v.at[b3]], elv.at[b2], esem).wait()
            pltpu.make_async_copy(
                er_h.at[didx_v.at[b3]], erv.at[b2], esem).wait()

            for g in range(G):
                sl = pl.ds(g * 16, 16)
                d16 = didx_v[b3, sl]
                z = elv[b2, sl] + erv[b2, sl]
                p16 = jnp.exp(jnp.where(z >= 0, z, z * 0.2))
                plsc.addupdate_scatter(den_v, [d16], p16)
                p_v[sl] = p16

            @pl.when(c >= 1)
            def _():
                wait_qscatters(1 - b2, lax.rem(c + 2, 3))

            @pl.when(c + 1 < NCK)
            def _():
                wait_idx()
                start_gathers(1 - b2, lax.rem(c + 1, 3))

            @pl.when(c + 2 < NCK)
            def _():
                start_idx(c + 2, lax.rem(c + 2, 3))

            pltpu.make_async_copy(
                feat_h.at[sidx_v.at[b3]],
                rows_v.at[pl.ds(b2 * CH, CH)], gsem).wait()

            for g in range(G):
                def scale(r, carry2):
                    row = b2 * CH + g * 16 + r
                    pr = plsc.load_gather(
                        p_v, [jnp.zeros((16,), jnp.int32) + (g * 16 + r)])
                    for j in range(D // 16):
                        sl = pl.ds(j * 16, 16)
                        rows_v[row, sl] = rows_v[row, sl] * pr
                    return carry2
                lax.fori_loop(0, 16, scale, 0)
                pltpu.make_async_copy(
                    rows_v.at[pl.ds(b2 * CH + g * 16, 16)],
                    acc_sh.at[didx_q.at[b3, g]], ssem).start(add=True)
            return carry
        lax.fori_loop(0, NCK, chunk, 0)
        wait_qscatters(lax.rem(NCK - 1, 2), lax.rem(NCK - 1, 3))

        plsc.subcore_barrier()

        @pl.when(sid < 15)
        def _():
            pltpu.sync_copy(acc_sh.at[pl.ds(sid * 624, 624)],
                            part_o.at[cid, pl.ds(sid * 624, 624)])

        @pl.when(sid == 15)
        def _():
            pltpu.sync_copy(acc_sh.at[pl.ds(15 * 624, N - 15 * 624)],
                            part_o.at[cid, pl.ds(15 * 624, N - 15 * 624)])

        pltpu.sync_copy(den_v, den_o.at[pl.ds(wid * N, N)])

    return k(feat, el_h_in, er_h_in, src_flat, dst_flat)


def kernel(h_inputs, objectives, edge_index, W1, al1, ar1, b1, W2, al2, ar2, b2):
    x = jnp.concatenate([h_inputs, objectives], axis=1)
    alr1 = jnp.stack([al1.reshape(-1), ar1.reshape(-1)], axis=1)
    alr2 = jnp.stack([al2.reshape(-1), ar2.reshape(-1)], axis=1)
    src_flat = edge_index[0]
    dst_flat = edge_index[1]

    feat1, ee1 = _tc_first(x, W1, alr1)
    part1, den1 = _sc_edge(feat1, ee1[:, 0], ee1[:, 1], src_flat, dst_flat)
    feat2, ee2 = _tc_mid(part1, den1.reshape(NT, N), b1.reshape(1, D), W2, alr2)
    part2, den2 = _sc_edge(feat2, ee2[:, 0], ee2[:, 1], src_flat, dst_flat)
    return _tc_final(part2, den2.reshape(NT, N), b2.reshape(1, D))


# speedup vs baseline: 42.2037x; 1.0250x over previous
import functools

import jax
import jax.numpy as jnp
from jax import lax
from jax.experimental import pallas as pl
from jax.experimental.pallas import tpu as pltpu
from jax.experimental.pallas import tpu_sc as plsc

N = 10000
E = 320000
D = 128

NT = 32
EPT = E // NT
CH = 80
NCK = EPT // CH
G = CH // 16
NPT = N // 16

_INTERPRET = False


def _tc_first(x, W, alr):
    BLK = 1280

    def body(x_ref, w_ref, alr_ref, feat_ref, ee_ref):
        feat = jnp.dot(x_ref[...], w_ref[...], preferred_element_type=jnp.float32)
        feat_ref[...] = feat
        ee_ref[...] = jnp.dot(feat, alr_ref[...], preferred_element_type=jnp.float32)

    return pl.pallas_call(
        body,
        grid=(pl.cdiv(N, BLK),),
        in_specs=[
            pl.BlockSpec((BLK, D), lambda i: (i, 0)),
            pl.BlockSpec((D, D), lambda i: (0, 0)),
            pl.BlockSpec((D, 2), lambda i: (0, 0)),
        ],
        out_specs=[
            pl.BlockSpec((BLK, D), lambda i: (i, 0)),
            pl.BlockSpec((BLK, 2), lambda i: (i, 0)),
        ],
        out_shape=[
            jax.ShapeDtypeStruct((N, D), jnp.float32),
            jax.ShapeDtypeStruct((N, 2), jnp.float32),
        ],
        interpret=_INTERPRET,
    )(x, W, alr)


def _combine(part_ref, den_ref, b_ref):
    dsum = jnp.sum(den_ref[...], axis=0)
    return (part_ref[0] + part_ref[1]) / (dsum[:, None] + 1e-9) + b_ref[...]


def _tc_mid(part, den, b, W, alr):
    BLK = 1280

    def body(part_ref, den_ref, b_ref, w_ref, alr_ref, feat_ref, ee_ref):
        h = _combine(part_ref, den_ref, b_ref)
        feat = jnp.dot(h, w_ref[...], preferred_element_type=jnp.float32)
        feat_ref[...] = feat
        ee_ref[...] = jnp.dot(feat, alr_ref[...], preferred_element_type=jnp.float32)

    return pl.pallas_call(
        body,
        grid=(pl.cdiv(N, BLK),),
        in_specs=[
            pl.BlockSpec((2, BLK, D), lambda i: (0, i, 0)),
            pl.BlockSpec((NT, BLK), lambda i: (0, i)),
            pl.BlockSpec((1, D), lambda i: (0, 0)),
            pl.BlockSpec((D, D), lambda i: (0, 0)),
            pl.BlockSpec((D, 2), lambda i: (0, 0)),
        ],
        out_specs=[
            pl.BlockSpec((BLK, D), lambda i: (i, 0)),
            pl.BlockSpec((BLK, 2), lambda i: (i, 0)),
        ],
        out_shape=[
            jax.ShapeDtypeStruct((N, D), jnp.float32),
            jax.ShapeDtypeStruct((N, 2), jnp.float32),
        ],
        interpret=_INTERPRET,
    )(part, den, b, W, alr)


def _tc_final(part, den, b):
    BLK = 1280

    def body(part_ref, den_ref, b_ref, h_ref):
        h_ref[...] = _combine(part_ref, den_ref, b_ref)

    return pl.pallas_call(
        body,
        grid=(pl.cdiv(N, BLK),),
        in_specs=[
            pl.BlockSpec((2, BLK, D), lambda i: (0, i, 0)),
            pl.BlockSpec((NT, BLK), lambda i: (0, i)),
            pl.BlockSpec((1, D), lambda i: (0, 0)),
        ],
        out_specs=pl.BlockSpec((BLK, D), lambda i: (i, 0)),
        out_shape=jax.ShapeDtypeStruct((N, D), jnp.float32),
        interpret=_INTERPRET,
    )(part, den, b)


def _sc_edge(feat, el_h_in, er_h_in, src_flat, dst_flat):
    mesh = plsc.VectorSubcoreMesh(core_axis_name="c", subcore_axis_name="s")

    @functools.partial(
        pl.kernel,
        out_type=[
            jax.ShapeDtypeStruct((2, N, D), jnp.float32),
            jax.ShapeDtypeStruct((NT * N,), jnp.float32),
        ],
        mesh=mesh,
        compiler_params=pltpu.CompilerParams(needs_layout_passes=False),
        scratch_types=[
            pltpu.VMEM((N,), jnp.float32),
            pltpu.VMEM((CH,), jnp.float32),
            pltpu.VMEM((2 * CH, D), jnp.float32),
            pltpu.VMEM((3, CH), jnp.int32),
            pltpu.VMEM((3, CH), jnp.int32),
            pltpu.VMEM((2, CH), jnp.float32),
            pltpu.VMEM((2, CH), jnp.float32),
            pltpu.VMEM((3, G, 16), jnp.int32),
            pltpu.VMEM_SHARED((N, D), jnp.float32),
            pltpu.SemaphoreType.DMA,
            pltpu.SemaphoreType.DMA,
            pltpu.SemaphoreType.DMA,
            pltpu.SemaphoreType.DMA,
        ],
        interpret=_INTERPRET,
    )
    def k(feat_h, el_h, er_h, src_h, dst_h, part_o, den_o,
          den_v, p_v, rows_v, sidx_v, didx_v, elv, erv, didx_q, acc_sh,
          isem, esem, gsem, ssem):
        cid = lax.axis_index("c")
        sid = lax.axis_index("s")
        wid = cid * 16 + sid
        base = wid * EPT

        z16 = jnp.zeros((16,), jnp.float32)

        def zden(i, carry):
            den_v[pl.ds(i * 16, 16)] = z16
            return carry
        lax.fori_loop(0, N // 16, zden, 0)

        def zblk(i, carry):
            for j in range(D // 16):
                rows_v[i, pl.ds(j * 16, 16)] = z16
            return carry
        lax.fori_loop(0, 125, zblk, 0)
        for t in range(NPT // 125):
            pltpu.sync_copy(rows_v.at[pl.ds(0, 125)],
                            acc_sh.at[pl.ds(sid * NPT + t * 125, 125)])
        plsc.subcore_barrier()

        def start_idx(c, slot):
            pltpu.make_async_copy(
                src_h.at[pl.ds(base + c * CH, CH)], sidx_v.at[slot], isem).start()
            pltpu.make_async_copy(
                dst_h.at[pl.ds(base + c * CH, CH)], didx_v.at[slot], isem).start()
            for g in range(G):
                pltpu.make_async_copy(
                    dst_h.at[pl.ds(base + c * CH + g * 16, 16)],
                    didx_q.at[slot, g], isem).start()

        def wait_idx():
            pltpu.make_async_copy(
                src_h.at[pl.ds(base, CH)], sidx_v.at[0], isem).wait()
            pltpu.make_async_copy(
                dst_h.at[pl.ds(base, CH)], didx_v.at[0], isem).wait()
            for g in range(G):
                pltpu.make_async_copy(
                    dst_h.at[pl.ds(base, 16)], didx_q.at[0, g], isem).wait()

        def start_gathers(b2, b3):
            pltpu.make_async_copy(
                el_h.at[sidx_v.at[b3]], elv.at[b2], esem).start()
            pltpu.make_async_copy(
                er_h.at[didx_v.at[b3]], erv.at[b2], esem).start()
            pltpu.make_async_copy(
                feat_h.at[sidx_v.at[b3]],
                rows_v.at[pl.ds(b2 * CH, CH)], gsem).start()

        start_idx(0, 0)
        start_idx(1, 1)
        wait_idx()
        start_gathers(0, 0)

        def wait_qscatters(b2, b3):
            for g in range(G):
                pltpu.make_async_copy(
                    rows_v.at[pl.ds(b2 * CH + g * 16, 16)],
                    acc_sh.at[didx_q.at[b3, g]], ssem).wait()

        def chunk(c, carry):
            b2 = lax.rem(c, 2)
            b3 = lax.rem(c, 3)

            pltpu.make_async_copy(
                el_h.at[sidx_v.at[b3]], elv.at[b2], esem).wait()
            pltpu.make_async_copy(
                er_h.at[didx_v.at[b3]], erv.at[b2], esem).wait()

            for g in range(G):
                sl = pl.ds(g * 16, 16)
                d16 = didx_v[b3, sl]
                z = elv[b2, sl] + erv[b2, sl]
                p16 = jnp.exp(jnp.where(z >= 0, z, z * 0.2))
                plsc.addupdate_scatter(den_v, [d16], p16)
                p_v[sl] = p16

            @pl.when(c >= 1)
            def _():
                wait_qscatters(1 - b2, lax.rem(c + 2, 3))

            @pl.when(c + 1 < NCK)
            def _():
                wait_idx()
                start_gathers(1 - b2, lax.rem(c + 1, 3))

            @pl.when(c + 2 < NCK)
            def _():
                start_idx(c + 2, lax.rem(c + 2, 3))

            pltpu.make_async_copy(
                feat_h.at[sidx_v.at[b3]],
                rows_v.at[pl.ds(b2 * CH, CH)], gsem).wait()

            for g in range(G):
                @plsc.parallel_loop(0, 16, unroll=4)
                def _(r):
                    row = b2 * CH + g * 16 + r
                    pr = plsc.load_gather(
                        p_v, [jnp.zeros((16,), jnp.int32) + (g * 16 + r)])
                    for j in range(D // 16):
                        sl = pl.ds(j * 16, 16)
                        rows_v[row, sl] = rows_v[row, sl] * pr
                pltpu.make_async_copy(
                    rows_v.at[pl.ds(b2 * CH + g * 16, 16)],
                    acc_sh.at[didx_q.at[b3, g]], ssem).start(add=True)
            return carry
        lax.fori_loop(0, NCK, chunk, 0)
        wait_qscatters(lax.rem(NCK - 1, 2), lax.rem(NCK - 1, 3))

        plsc.subcore_barrier()

        @pl.when(sid < 15)
        def _():
            pltpu.sync_copy(acc_sh.at[pl.ds(sid * 624, 624)],
                            part_o.at[cid, pl.ds(sid * 624, 624)])

        @pl.when(sid == 15)
        def _():
            pltpu.sync_copy(acc_sh.at[pl.ds(15 * 624, N - 15 * 624)],
                            part_o.at[cid, pl.ds(15 * 624, N - 15 * 624)])

        pltpu.sync_copy(den_v, den_o.at[pl.ds(wid * N, N)])

    return k(feat, el_h_in, er_h_in, src_flat, dst_flat)


def kernel(h_inputs, objectives, edge_index, W1, al1, ar1, b1, W2, al2, ar2, b2):
    x = jnp.concatenate([h_inputs, objectives], axis=1)
    alr1 = jnp.stack([al1.reshape(-1), ar1.reshape(-1)], axis=1)
    alr2 = jnp.stack([al2.reshape(-1), ar2.reshape(-1)], axis=1)
    src_flat = edge_index[0]
    dst_flat = edge_index[1]

    feat1, ee1 = _tc_first(x, W1, alr1)
    part1, den1 = _sc_edge(feat1, ee1[:, 0], ee1[:, 1], src_flat, dst_flat)
    feat2, ee2 = _tc_mid(part1, den1.reshape(NT, N), b1.reshape(1, D), W2, alr2)
    part2, den2 = _sc_edge(feat2, ee2[:, 0], ee2[:, 1], src_flat, dst_flat)
    return _tc_final(part2, den2.reshape(NT, N), b2.reshape(1, D))


# speedup vs baseline: 42.5240x; 1.0076x over previous
import functools

import jax
import jax.numpy as jnp
from jax import lax
from jax.experimental import pallas as pl
from jax.experimental.pallas import tpu as pltpu
from jax.experimental.pallas import tpu_sc as plsc

N = 10000
E = 320000
D = 128

NT = 32
EPT = E // NT
CH = 80
NCK = EPT // CH
G = CH // 16
NPT = N // 16

_INTERPRET = False


def _tc_first(x, W, alr):
    BLK = 1280

    def body(x_ref, w_ref, alr_ref, feat_ref, ee_ref):
        feat = jnp.dot(x_ref[...], w_ref[...], preferred_element_type=jnp.float32)
        feat_ref[...] = feat
        ee_ref[...] = jnp.dot(feat, alr_ref[...], preferred_element_type=jnp.float32)

    return pl.pallas_call(
        body,
        grid=(pl.cdiv(N, BLK),),
        in_specs=[
            pl.BlockSpec((BLK, D), lambda i: (i, 0)),
            pl.BlockSpec((D, D), lambda i: (0, 0)),
            pl.BlockSpec((D, 2), lambda i: (0, 0)),
        ],
        out_specs=[
            pl.BlockSpec((BLK, D), lambda i: (i, 0)),
            pl.BlockSpec((BLK, 2), lambda i: (i, 0)),
        ],
        out_shape=[
            jax.ShapeDtypeStruct((N, D), jnp.float32),
            jax.ShapeDtypeStruct((N, 2), jnp.float32),
        ],
        interpret=_INTERPRET,
    )(x, W, alr)


def _combine(part_ref, den_ref, b_ref):
    dsum = jnp.sum(den_ref[...], axis=0)
    return (part_ref[0] + part_ref[1]) / (dsum[:, None] + 1e-9) + b_ref[...]


def _tc_mid(part, den, b, W, alr):
    BLK = 1280

    def body(part_ref, den_ref, b_ref, w_ref, alr_ref, feat_ref, ee_ref):
        h = _combine(part_ref, den_ref, b_ref)
        feat = jnp.dot(h, w_ref[...], preferred_element_type=jnp.float32)
        feat_ref[...] = feat
        ee_ref[...] = jnp.dot(feat, alr_ref[...], preferred_element_type=jnp.float32)

    return pl.pallas_call(
        body,
        grid=(pl.cdiv(N, BLK),),
        in_specs=[
            pl.BlockSpec((2, BLK, D), lambda i: (0, i, 0)),
            pl.BlockSpec((NT, BLK), lambda i: (0, i)),
            pl.BlockSpec((1, D), lambda i: (0, 0)),
            pl.BlockSpec((D, D), lambda i: (0, 0)),
            pl.BlockSpec((D, 2), lambda i: (0, 0)),
        ],
        out_specs=[
            pl.BlockSpec((BLK, D), lambda i: (i, 0)),
            pl.BlockSpec((BLK, 2), lambda i: (i, 0)),
        ],
        out_shape=[
            jax.ShapeDtypeStruct((N, D), jnp.float32),
            jax.ShapeDtypeStruct((N, 2), jnp.float32),
        ],
        interpret=_INTERPRET,
    )(part, den, b, W, alr)


def _tc_final(part, den, b):
    BLK = 1280

    def body(part_ref, den_ref, b_ref, h_ref):
        h_ref[...] = _combine(part_ref, den_ref, b_ref)

    return pl.pallas_call(
        body,
        grid=(pl.cdiv(N, BLK),),
        in_specs=[
            pl.BlockSpec((2, BLK, D), lambda i: (0, i, 0)),
            pl.BlockSpec((NT, BLK), lambda i: (0, i)),
            pl.BlockSpec((1, D), lambda i: (0, 0)),
        ],
        out_specs=pl.BlockSpec((BLK, D), lambda i: (i, 0)),
        out_shape=jax.ShapeDtypeStruct((N, D), jnp.float32),
        interpret=_INTERPRET,
    )(part, den, b)


def _sc_edge(feat, el_h_in, er_h_in, src_flat, dst_flat):
    mesh = plsc.VectorSubcoreMesh(core_axis_name="c", subcore_axis_name="s")

    @functools.partial(
        pl.kernel,
        out_type=[
            jax.ShapeDtypeStruct((2, N, D), jnp.float32),
            jax.ShapeDtypeStruct((NT * N,), jnp.float32),
        ],
        mesh=mesh,
        compiler_params=pltpu.CompilerParams(needs_layout_passes=False),
        scratch_types=[
            pltpu.VMEM((N,), jnp.float32),
            pltpu.VMEM((CH,), jnp.float32),
            pltpu.VMEM((2 * CH, D), jnp.float32),
            pltpu.VMEM((3, CH), jnp.int32),
            pltpu.VMEM((3, CH), jnp.int32),
            pltpu.VMEM((2, CH), jnp.float32),
            pltpu.VMEM((2, CH), jnp.float32),
            pltpu.VMEM((3, G, 16), jnp.int32),
            pltpu.VMEM_SHARED((N, D), jnp.float32),
            pltpu.SemaphoreType.DMA,
            pltpu.SemaphoreType.DMA,
            pltpu.SemaphoreType.DMA,
            pltpu.SemaphoreType.DMA,
        ],
        interpret=_INTERPRET,
    )
    def k(feat_h, el_h, er_h, src_h, dst_h, part_o, den_o,
          den_v, p_v, rows_v, sidx_v, didx_v, elv, erv, didx_q, acc_sh,
          isem, esem, gsem, ssem):
        cid = lax.axis_index("c")
        sid = lax.axis_index("s")
        wid = cid * 16 + sid
        base = wid * EPT

        z16 = jnp.zeros((16,), jnp.float32)

        def zden(i, carry):
            den_v[pl.ds(i * 16, 16)] = z16
            return carry
        lax.fori_loop(0, N // 16, zden, 0)

        def zblk(i, carry):
            for j in range(D // 16):
                rows_v[i, pl.ds(j * 16, 16)] = z16
            return carry
        lax.fori_loop(0, 125, zblk, 0)
        for t in range(NPT // 125):
            pltpu.sync_copy(rows_v.at[pl.ds(0, 125)],
                            acc_sh.at[pl.ds(sid * NPT + t * 125, 125)])
        plsc.subcore_barrier()

        def start_idx(c, slot):
            pltpu.make_async_copy(
                src_h.at[pl.ds(base + c * CH, CH)], sidx_v.at[slot], isem).start()
            pltpu.make_async_copy(
                dst_h.at[pl.ds(base + c * CH, CH)], didx_v.at[slot], isem).start()
            for g in range(G):
                pltpu.make_async_copy(
                    dst_h.at[pl.ds(base + c * CH + g * 16, 16)],
                    didx_q.at[slot, g], isem).start()

        def wait_idx():
            pltpu.make_async_copy(
                src_h.at[pl.ds(base, CH)], sidx_v.at[0], isem).wait()
            pltpu.make_async_copy(
                dst_h.at[pl.ds(base, CH)], didx_v.at[0], isem).wait()
            for g in range(G):
                pltpu.make_async_copy(
                    dst_h.at[pl.ds(base, 16)], didx_q.at[0, g], isem).wait()

        def start_gathers(b2, b3):
            pltpu.make_async_copy(
                el_h.at[sidx_v.at[b3]], elv.at[b2], esem).start()
            pltpu.make_async_copy(
                er_h.at[didx_v.at[b3]], erv.at[b2], esem).start()
            pltpu.make_async_copy(
                feat_h.at[sidx_v.at[b3]],
                rows_v.at[pl.ds(b2 * CH, CH)], gsem).start()

        start_idx(0, 0)
        start_idx(1, 1)
        wait_idx()
        start_gathers(0, 0)

        def wait_qscatters(b2, b3):
            for g in range(G):
                pltpu.make_async_copy(
                    rows_v.at[pl.ds(b2 * CH + g * 16, 16)],
                    acc_sh.at[didx_q.at[b3, g]], ssem).wait()

        def chunk(c, carry):
            b2 = lax.rem(c, 2)
            b3 = lax.rem(c, 3)

            pltpu.make_async_copy(
                el_h.at[sidx_v.at[b3]], elv.at[b2], esem).wait()
            pltpu.make_async_copy(
                er_h.at[didx_v.at[b3]], erv.at[b2], esem).wait()

            ps = []
            for g in range(G):
                sl = pl.ds(g * 16, 16)
                d16 = didx_v[b3, sl]
                z = elv[b2, sl] + erv[b2, sl]
                p16 = jnp.exp(jnp.where(z >= 0, z, z * 0.2))
                plsc.addupdate_scatter(den_v, [d16], p16)
                ps.append(p16)

            @pl.when(c >= 1)
            def _():
                wait_qscatters(1 - b2, lax.rem(c + 2, 3))

            @pl.when(c + 1 < NCK)
            def _():
                wait_idx()
                start_gathers(1 - b2, lax.rem(c + 1, 3))

            @pl.when(c + 2 < NCK)
            def _():
                start_idx(c + 2, lax.rem(c + 2, 3))

            pltpu.make_async_copy(
                feat_h.at[sidx_v.at[b3]],
                rows_v.at[pl.ds(b2 * CH, CH)], gsem).wait()

            for g in range(G):
                @plsc.parallel_loop(0, 16, unroll=4)
                def _(r):
                    row = b2 * CH + g * 16 + r
                    _p = ps[g]
                    pr = _p[jnp.zeros((16,), jnp.int32) + r]
                    for j in range(D // 16):
                        sl = pl.ds(j * 16, 16)
                        rows_v[row, sl] = rows_v[row, sl] * pr
                pltpu.make_async_copy(
                    rows_v.at[pl.ds(b2 * CH + g * 16, 16)],
                    acc_sh.at[didx_q.at[b3, g]], ssem).start(add=True)
            return carry
        lax.fori_loop(0, NCK, chunk, 0)
        wait_qscatters(lax.rem(NCK - 1, 2), lax.rem(NCK - 1, 3))

        plsc.subcore_barrier()

        @pl.when(sid < 15)
        def _():
            pltpu.sync_copy(acc_sh.at[pl.ds(sid * 624, 624)],
                            part_o.at[cid, pl.ds(sid * 624, 624)])

        @pl.when(sid == 15)
        def _():
            pltpu.sync_copy(acc_sh.at[pl.ds(15 * 624, N - 15 * 624)],
                            part_o.at[cid, pl.ds(15 * 624, N - 15 * 624)])

        pltpu.sync_copy(den_v, den_o.at[pl.ds(wid * N, N)])

    return k(feat, el_h_in, er_h_in, src_flat, dst_flat)


def kernel(h_inputs, objectives, edge_index, W1, al1, ar1, b1, W2, al2, ar2, b2):
    x = jnp.concatenate([h_inputs, objectives], axis=1)
    alr1 = jnp.stack([al1.reshape(-1), ar1.reshape(-1)], axis=1)
    alr2 = jnp.stack([al2.reshape(-1), ar2.reshape(-1)], axis=1)
    src_flat = edge_index[0]
    dst_flat = edge_index[1]

    feat1, ee1 = _tc_first(x, W1, alr1)
    part1, den1 = _sc_edge(feat1, ee1[:, 0], ee1[:, 1], src_flat, dst_flat)
    feat2, ee2 = _tc_mid(part1, den1.reshape(NT, N), b1.reshape(1, D), W2, alr2)
    part2, den2 = _sc_edge(feat2, ee2[:, 0], ee2[:, 1], src_flat, dst_flat)
    return _tc_final(part2, den2.reshape(NT, N), b2.reshape(1, D))


# speedup vs baseline: 42.9859x; 1.0109x over previous
import functools

import jax
import jax.numpy as jnp
from jax import lax
from jax.experimental import pallas as pl
from jax.experimental.pallas import tpu as pltpu
from jax.experimental.pallas import tpu_sc as plsc

N = 10000
E = 320000
D = 128

NT = 32
EPT = E // NT
CH = 80
NCK = EPT // CH
G = CH // 16
NPT = N // 16

_INTERPRET = False


def _tc_first(h_in, obj, W, alr):
    BLK = 1280

    def body(h_ref, o_ref, w_ref, alr_ref, feat_ref, ee_ref):
        w = w_ref[...]
        feat = jnp.dot(h_ref[...], w[:D - 1, :], preferred_element_type=jnp.float32)
        feat = feat + o_ref[...] * w[D - 1:D, :]
        feat_ref[...] = feat
        ee_ref[...] = jnp.dot(feat, alr_ref[...], preferred_element_type=jnp.float32)

    return pl.pallas_call(
        body,
        grid=(pl.cdiv(N, BLK),),
        in_specs=[
            pl.BlockSpec((BLK, D - 1), lambda i: (i, 0)),
            pl.BlockSpec((BLK, 1), lambda i: (i, 0)),
            pl.BlockSpec((D, D), lambda i: (0, 0)),
            pl.BlockSpec((D, 2), lambda i: (0, 0)),
        ],
        out_specs=[
            pl.BlockSpec((BLK, D), lambda i: (i, 0)),
            pl.BlockSpec((BLK, 2), lambda i: (i, 0)),
        ],
        out_shape=[
            jax.ShapeDtypeStruct((N, D), jnp.float32),
            jax.ShapeDtypeStruct((N, 2), jnp.float32),
        ],
        interpret=_INTERPRET,
    )(h_in, obj, W, alr)


def _combine(part_ref, den_ref, b_ref):
    dsum = jnp.sum(den_ref[...], axis=0)
    return (part_ref[0] + part_ref[1]) / (dsum[:, None] + 1e-9) + b_ref[...]


def _tc_mid(part, den, b, W, alr):
    BLK = 1280

    def body(part_ref, den_ref, b_ref, w_ref, alr_ref, feat_ref, ee_ref):
        h = _combine(part_ref, den_ref, b_ref)
        feat = jnp.dot(h, w_ref[...], preferred_element_type=jnp.float32)
        feat_ref[...] = feat
        ee_ref[...] = jnp.dot(feat, alr_ref[...], preferred_element_type=jnp.float32)

    return pl.pallas_call(
        body,
        grid=(pl.cdiv(N, BLK),),
        in_specs=[
            pl.BlockSpec((2, BLK, D), lambda i: (0, i, 0)),
            pl.BlockSpec((NT, BLK), lambda i: (0, i)),
            pl.BlockSpec((1, D), lambda i: (0, 0)),
            pl.BlockSpec((D, D), lambda i: (0, 0)),
            pl.BlockSpec((D, 2), lambda i: (0, 0)),
        ],
        out_specs=[
            pl.BlockSpec((BLK, D), lambda i: (i, 0)),
            pl.BlockSpec((BLK, 2), lambda i: (i, 0)),
        ],
        out_shape=[
            jax.ShapeDtypeStruct((N, D), jnp.float32),
            jax.ShapeDtypeStruct((N, 2), jnp.float32),
        ],
        interpret=_INTERPRET,
    )(part, den, b, W, alr)


def _tc_final(part, den, b):
    BLK = 1280

    def body(part_ref, den_ref, b_ref, h_ref):
        h_ref[...] = _combine(part_ref, den_ref, b_ref)

    return pl.pallas_call(
        body,
        grid=(pl.cdiv(N, BLK),),
        in_specs=[
            pl.BlockSpec((2, BLK, D), lambda i: (0, i, 0)),
            pl.BlockSpec((NT, BLK), lambda i: (0, i)),
            pl.BlockSpec((1, D), lambda i: (0, 0)),
        ],
        out_specs=pl.BlockSpec((BLK, D), lambda i: (i, 0)),
        out_shape=jax.ShapeDtypeStruct((N, D), jnp.float32),
        interpret=_INTERPRET,
    )(part, den, b)


def _sc_edge(feat, el_h_in, er_h_in, src_flat, dst_flat):
    mesh = plsc.VectorSubcoreMesh(core_axis_name="c", subcore_axis_name="s")

    @functools.partial(
        pl.kernel,
        out_type=[
            jax.ShapeDtypeStruct((2, N, D), jnp.float32),
            jax.ShapeDtypeStruct((NT * N,), jnp.float32),
        ],
        mesh=mesh,
        compiler_params=pltpu.CompilerParams(needs_layout_passes=False),
        scratch_types=[
            pltpu.VMEM((N,), jnp.float32),
            pltpu.VMEM((CH,), jnp.float32),
            pltpu.VMEM((2 * CH, D), jnp.float32),
            pltpu.VMEM((3, CH), jnp.int32),
            pltpu.VMEM((3, CH), jnp.int32),
            pltpu.VMEM((2, CH), jnp.float32),
            pltpu.VMEM((2, CH), jnp.float32),
            pltpu.VMEM((3, 32), jnp.int32),
            pltpu.VMEM((3, 48), jnp.int32),
            pltpu.VMEM_SHARED((N, D), jnp.float32),
            pltpu.SemaphoreType.DMA,
            pltpu.SemaphoreType.DMA,
            pltpu.SemaphoreType.DMA,
            pltpu.SemaphoreType.DMA,
        ],
        interpret=_INTERPRET,
    )
    def k(feat_h, el_h, er_h, src_h, dst_h, part_o, den_o,
          den_v, p_v, rows_v, sidx_v, didx_v, elv, erv, didx_a, didx_b, acc_sh,
          isem, esem, gsem, ssem):
        cid = lax.axis_index("c")
        sid = lax.axis_index("s")
        wid = cid * 16 + sid
        base = wid * EPT

        z16 = jnp.zeros((16,), jnp.float32)

        def zden(i, carry):
            den_v[pl.ds(i * 16, 16)] = z16
            return carry
        lax.fori_loop(0, N // 16, zden, 0)

        def zblk(i, carry):
            for j in range(D // 16):
                rows_v[i, pl.ds(j * 16, 16)] = z16
            return carry
        lax.fori_loop(0, 125, zblk, 0)
        for t in range(NPT // 125):
            pltpu.sync_copy(rows_v.at[pl.ds(0, 125)],
                            acc_sh.at[pl.ds(sid * NPT + t * 125, 125)])
        plsc.subcore_barrier()

        def start_idx(c, slot):
            pltpu.make_async_copy(
                src_h.at[pl.ds(base + c * CH, CH)], sidx_v.at[slot], isem).start()
            pltpu.make_async_copy(
                dst_h.at[pl.ds(base + c * CH, CH)], didx_v.at[slot], isem).start()
            pltpu.make_async_copy(
                dst_h.at[pl.ds(base + c * CH, 32)], didx_a.at[slot], isem).start()
            pltpu.make_async_copy(
                dst_h.at[pl.ds(base + c * CH + 32, 48)], didx_b.at[slot], isem).start()

        def wait_idx():
            pltpu.make_async_copy(
                src_h.at[pl.ds(base, CH)], sidx_v.at[0], isem).wait()
            pltpu.make_async_copy(
                dst_h.at[pl.ds(base, CH)], didx_v.at[0], isem).wait()
            pltpu.make_async_copy(
                dst_h.at[pl.ds(base, 32)], didx_a.at[0], isem).wait()
            pltpu.make_async_copy(
                dst_h.at[pl.ds(base, 48)], didx_b.at[0], isem).wait()

        def start_gathers(b2, b3):
            pltpu.make_async_copy(
                el_h.at[sidx_v.at[b3]], elv.at[b2], esem).start()
            pltpu.make_async_copy(
                er_h.at[didx_v.at[b3]], erv.at[b2], esem).start()
            pltpu.make_async_copy(
                feat_h.at[sidx_v.at[b3]],
                rows_v.at[pl.ds(b2 * CH, CH)], gsem).start()

        start_idx(0, 0)
        start_idx(1, 1)
        wait_idx()
        start_gathers(0, 0)

        def wait_qscatters(b2, b3):
            pltpu.make_async_copy(
                rows_v.at[pl.ds(b2 * CH, 32)],
                acc_sh.at[didx_a.at[b3]], ssem).wait()
            pltpu.make_async_copy(
                rows_v.at[pl.ds(b2 * CH + 32, 48)],
                acc_sh.at[didx_b.at[b3]], ssem).wait()

        def chunk(c, carry):
            b2 = lax.rem(c, 2)
            b3 = lax.rem(c, 3)

            pltpu.make_async_copy(
                el_h.at[sidx_v.at[b3]], elv.at[b2], esem).wait()
            pltpu.make_async_copy(
                er_h.at[didx_v.at[b3]], erv.at[b2], esem).wait()

            ps = []
            for g in range(G):
                sl = pl.ds(g * 16, 16)
                d16 = didx_v[b3, sl]
                z = elv[b2, sl] + erv[b2, sl]
                p16 = jnp.exp(jnp.where(z >= 0, z, z * 0.2))
                plsc.addupdate_scatter(den_v, [d16], p16)
                ps.append(p16)

            @pl.when(c >= 1)
            def _():
                wait_qscatters(1 - b2, lax.rem(c + 2, 3))

            @pl.when(c + 1 < NCK)
            def _():
                wait_idx()
                start_gathers(1 - b2, lax.rem(c + 1, 3))

            @pl.when(c + 2 < NCK)
            def _():
                start_idx(c + 2, lax.rem(c + 2, 3))

            pltpu.make_async_copy(
                feat_h.at[sidx_v.at[b3]],
                rows_v.at[pl.ds(b2 * CH, CH)], gsem).wait()

            for g_lo, g_hi, idx_ring in ((0, 2, didx_a), (2, G, didx_b)):
                for g in range(g_lo, g_hi):
                    @plsc.parallel_loop(0, 16, unroll=4)
                    def _(r):
                        row = b2 * CH + g * 16 + r
                        _p = ps[g]
                        pr = _p[jnp.zeros((16,), jnp.int32) + r]
                        for j in range(D // 16):
                            sl = pl.ds(j * 16, 16)
                            rows_v[row, sl] = rows_v[row, sl] * pr
                pltpu.make_async_copy(
                    rows_v.at[pl.ds(b2 * CH + g_lo * 16, (g_hi - g_lo) * 16)],
                    acc_sh.at[idx_ring.at[b3]], ssem).start(add=True)
            return carry
        lax.fori_loop(0, NCK, chunk, 0)
        wait_qscatters(lax.rem(NCK - 1, 2), lax.rem(NCK - 1, 3))

        plsc.subcore_barrier()

        @pl.when(sid < 15)
        def _():
            pltpu.sync_copy(acc_sh.at[pl.ds(sid * 624, 624)],
                            part_o.at[cid, pl.ds(sid * 624, 624)])

        @pl.when(sid == 15)
        def _():
            pltpu.sync_copy(acc_sh.at[pl.ds(15 * 624, N - 15 * 624)],
                            part_o.at[cid, pl.ds(15 * 624, N - 15 * 624)])

        pltpu.sync_copy(den_v, den_o.at[pl.ds(wid * N, N)])

    return k(feat, el_h_in, er_h_in, src_flat, dst_flat)


def kernel(h_inputs, objectives, edge_index, W1, al1, ar1, b1, W2, al2, ar2, b2):
    alr1 = jnp.stack([al1.reshape(-1), ar1.reshape(-1)], axis=1)
    alr2 = jnp.stack([al2.reshape(-1), ar2.reshape(-1)], axis=1)
    src_flat = edge_index[0]
    dst_flat = edge_index[1]

    feat1, ee1 = _tc_first(h_inputs, objectives, W1, alr1)
    part1, den1 = _sc_edge(feat1, ee1[:, 0], ee1[:, 1], src_flat, dst_flat)
    feat2, ee2 = _tc_mid(part1, den1.reshape(NT, N), b1.reshape(1, D), W2, alr2)
    part2, den2 = _sc_edge(feat2, ee2[:, 0], ee2[:, 1], src_flat, dst_flat)
    return _tc_final(part2, den2.reshape(NT, N), b2.reshape(1, D))


# speedup vs baseline: 43.0318x; 1.0011x over previous
import functools

import jax
import jax.numpy as jnp
from jax import lax
from jax.experimental import pallas as pl
from jax.experimental.pallas import tpu as pltpu
from jax.experimental.pallas import tpu_sc as plsc

N = 10000
E = 320000
D = 128

NT = 32
EPT = E // NT
CH = 80
NCK = EPT // CH
G = CH // 16
NPT = N // 16

_INTERPRET = False


def _tc_first(h_in, obj, W, alr):
    BLK = 1280

    def body(h_ref, o_ref, w_ref, alr_ref, feat_ref, ee_ref):
        w = w_ref[...]
        feat = jnp.dot(h_ref[...], w[:D - 1, :], preferred_element_type=jnp.float32)
        feat = feat + o_ref[...] * w[D - 1:D, :]
        feat_ref[...] = feat
        ee_ref[...] = jnp.dot(feat, alr_ref[...], preferred_element_type=jnp.float32)

    return pl.pallas_call(
        body,
        grid=(pl.cdiv(N, BLK),),
        in_specs=[
            pl.BlockSpec((BLK, D - 1), lambda i: (i, 0)),
            pl.BlockSpec((BLK, 1), lambda i: (i, 0)),
            pl.BlockSpec((D, D), lambda i: (0, 0)),
            pl.BlockSpec((D, 2), lambda i: (0, 0)),
        ],
        out_specs=[
            pl.BlockSpec((BLK, D), lambda i: (i, 0)),
            pl.BlockSpec((BLK, 2), lambda i: (i, 0)),
        ],
        out_shape=[
            jax.ShapeDtypeStruct((N, D), jnp.float32),
            jax.ShapeDtypeStruct((N, 2), jnp.float32),
        ],
        interpret=_INTERPRET,
    )(h_in, obj, W, alr)


def _combine(part_ref, den_ref, b_ref):
    dsum = jnp.sum(den_ref[...], axis=0)
    return (part_ref[0] + part_ref[1]) / (dsum[:, None] + 1e-9) + b_ref[...]


def _tc_mid(part, den, b, W, alr):
    BLK = 1280

    def body(part_ref, den_ref, b_ref, w_ref, alr_ref, feat_ref, ee_ref):
        h = _combine(part_ref, den_ref, b_ref)
        feat = jnp.dot(h, w_ref[...], preferred_element_type=jnp.float32)
        feat_ref[...] = feat
        ee_ref[...] = jnp.dot(feat, alr_ref[...], preferred_element_type=jnp.float32)

    return pl.pallas_call(
        body,
        grid=(pl.cdiv(N, BLK),),
        in_specs=[
            pl.BlockSpec((2, BLK, D), lambda i: (0, i, 0)),
            pl.BlockSpec((NT, BLK), lambda i: (0, i)),
            pl.BlockSpec((1, D), lambda i: (0, 0)),
            pl.BlockSpec((D, D), lambda i: (0, 0)),
            pl.BlockSpec((D, 2), lambda i: (0, 0)),
        ],
        out_specs=[
            pl.BlockSpec((BLK, D), lambda i: (i, 0)),
            pl.BlockSpec((BLK, 2), lambda i: (i, 0)),
        ],
        out_shape=[
            jax.ShapeDtypeStruct((N, D), jnp.float32),
            jax.ShapeDtypeStruct((N, 2), jnp.float32),
        ],
        interpret=_INTERPRET,
    )(part, den, b, W, alr)


def _tc_final(part, den, b):
    BLK = 1280

    def body(part_ref, den_ref, b_ref, h_ref):
        h_ref[...] = _combine(part_ref, den_ref, b_ref)

    return pl.pallas_call(
        body,
        grid=(pl.cdiv(N, BLK),),
        in_specs=[
            pl.BlockSpec((2, BLK, D), lambda i: (0, i, 0)),
            pl.BlockSpec((NT, BLK), lambda i: (0, i)),
            pl.BlockSpec((1, D), lambda i: (0, 0)),
        ],
        out_specs=pl.BlockSpec((BLK, D), lambda i: (i, 0)),
        out_shape=jax.ShapeDtypeStruct((N, D), jnp.float32),
        interpret=_INTERPRET,
    )(part, den, b)


def _sc_edge(feat, el_h_in, er_h_in, src_flat, dst_flat):
    mesh = plsc.VectorSubcoreMesh(core_axis_name="c", subcore_axis_name="s")

    @functools.partial(
        pl.kernel,
        out_type=[
            jax.ShapeDtypeStruct((2, N, D), jnp.float32),
            jax.ShapeDtypeStruct((NT * N,), jnp.float32),
        ],
        mesh=mesh,
        compiler_params=pltpu.CompilerParams(needs_layout_passes=False),
        scratch_types=[
            pltpu.VMEM((N,), jnp.float32),
            pltpu.VMEM((CH,), jnp.float32),
            pltpu.VMEM((2 * CH, D), jnp.float32),
            pltpu.VMEM((3, CH), jnp.int32),
            pltpu.VMEM((3, CH), jnp.int32),
            pltpu.VMEM((2, CH), jnp.float32),
            pltpu.VMEM((2, CH), jnp.float32),
            pltpu.VMEM((3, 32), jnp.int32),
            pltpu.VMEM((3, 48), jnp.int32),
            pltpu.VMEM_SHARED((N, D), jnp.float32),
            pltpu.SemaphoreType.DMA,
            pltpu.SemaphoreType.DMA,
            pltpu.SemaphoreType.DMA,
            pltpu.SemaphoreType.DMA,
        ],
        interpret=_INTERPRET,
    )
    def k(feat_h, el_h, er_h, src_h, dst_h, part_o, den_o,
          den_v, p_v, rows_v, sidx_v, didx_v, elv, erv, didx_a, didx_b, acc_sh,
          isem, esem, gsem, ssem):
        cid = lax.axis_index("c")
        sid = lax.axis_index("s")
        wid = cid * 16 + sid
        base = wid * EPT

        z16 = jnp.zeros((16,), jnp.float32)

        def zden(i, carry):
            den_v[pl.ds(i * 16, 16)] = z16
            return carry
        lax.fori_loop(0, N // 16, zden, 0)

        def zblk(i, carry):
            for j in range(D // 16):
                rows_v[i, pl.ds(j * 16, 16)] = z16
            return carry
        lax.fori_loop(0, 125, zblk, 0)
        for t in range(NPT // 125):
            pltpu.sync_copy(rows_v.at[pl.ds(0, 125)],
                            acc_sh.at[pl.ds(sid * NPT + t * 125, 125)])
        plsc.subcore_barrier()

        def start_idx(c, slot):
            pltpu.make_async_copy(
                src_h.at[pl.ds(base + c * CH, CH)], sidx_v.at[slot], isem).start()
            pltpu.make_async_copy(
                dst_h.at[pl.ds(base + c * CH, CH)], didx_v.at[slot], isem).start()
            pltpu.make_async_copy(
                dst_h.at[pl.ds(base + c * CH, 32)], didx_a.at[slot], isem).start()
            pltpu.make_async_copy(
                dst_h.at[pl.ds(base + c * CH + 32, 48)], didx_b.at[slot], isem).start()

        def wait_idx():
            pltpu.make_async_copy(
                src_h.at[pl.ds(base, CH)], sidx_v.at[0], isem).wait()
            pltpu.make_async_copy(
                dst_h.at[pl.ds(base, CH)], didx_v.at[0], isem).wait()
            pltpu.make_async_copy(
                dst_h.at[pl.ds(base, 32)], didx_a.at[0], isem).wait()
            pltpu.make_async_copy(
                dst_h.at[pl.ds(base, 48)], didx_b.at[0], isem).wait()

        def start_gathers(b2, b3):
            pltpu.make_async_copy(
                el_h.at[sidx_v.at[b3]], elv.at[b2], esem).start()
            pltpu.make_async_copy(
                er_h.at[didx_v.at[b3]], erv.at[b2], esem).start()
            pltpu.make_async_copy(
                feat_h.at[sidx_v.at[b3]],
                rows_v.at[pl.ds(b2 * CH, CH)], gsem).start()

        start_idx(0, 0)
        start_idx(1, 1)
        wait_idx()
        start_gathers(0, 0)

        def wait_qscatters(b2, b3):
            pltpu.make_async_copy(
                rows_v.at[pl.ds(b2 * CH, 32)],
                acc_sh.at[didx_a.at[b3]], ssem).wait()
            pltpu.make_async_copy(
                rows_v.at[pl.ds(b2 * CH + 32, 48)],
                acc_sh.at[didx_b.at[b3]], ssem).wait()

        def chunk(c, carry):
            b2 = lax.rem(c, 2)
            b3 = lax.rem(c, 3)

            pltpu.make_async_copy(
                el_h.at[sidx_v.at[b3]], elv.at[b2], esem).wait()
            pltpu.make_async_copy(
                er_h.at[didx_v.at[b3]], erv.at[b2], esem).wait()

            ps = []
            for g in range(G):
                sl = pl.ds(g * 16, 16)
                d16 = didx_v[b3, sl]
                z = elv[b2, sl] + erv[b2, sl]
                p16 = jnp.exp(jnp.where(z >= 0, z, z * 0.2))
                plsc.addupdate_scatter(den_v, [d16], p16)
                ps.append(p16)

            @pl.when(c >= 1)
            def _():
                wait_qscatters(1 - b2, lax.rem(c + 2, 3))

            @pl.when(c + 1 < NCK)
            def _():
                wait_idx()
                start_gathers(1 - b2, lax.rem(c + 1, 3))

            @pl.when(c + 2 < NCK)
            def _():
                start_idx(c + 2, lax.rem(c + 2, 3))

            pltpu.make_async_copy(
                feat_h.at[sidx_v.at[b3]],
                rows_v.at[pl.ds(b2 * CH, CH)], gsem).wait()

            for g_lo, g_hi, idx_ring in ((0, 2, didx_a), (2, G, didx_b)):
                for g in range(g_lo, g_hi):
                    @plsc.parallel_loop(0, 16, unroll=8)
                    def _(r):
                        row = b2 * CH + g * 16 + r
                        _p = ps[g]
                        pr = _p[jnp.zeros((16,), jnp.int32) + r]
                        for j in range(D // 16):
                            sl = pl.ds(j * 16, 16)
                            rows_v[row, sl] = rows_v[row, sl] * pr
                pltpu.make_async_copy(
                    rows_v.at[pl.ds(b2 * CH + g_lo * 16, (g_hi - g_lo) * 16)],
                    acc_sh.at[idx_ring.at[b3]], ssem).start(add=True)
            return carry
        lax.fori_loop(0, NCK, chunk, 0)
        wait_qscatters(lax.rem(NCK - 1, 2), lax.rem(NCK - 1, 3))

        plsc.subcore_barrier()

        @pl.when(sid < 15)
        def _():
            pltpu.sync_copy(acc_sh.at[pl.ds(sid * 624, 624)],
                            part_o.at[cid, pl.ds(sid * 624, 624)])

        @pl.when(sid == 15)
        def _():
            pltpu.sync_copy(acc_sh.at[pl.ds(15 * 624, N - 15 * 624)],
                            part_o.at[cid, pl.ds(15 * 624, N - 15 * 624)])

        pltpu.sync_copy(den_v, den_o.at[pl.ds(wid * N, N)])

    return k(feat, el_h_in, er_h_in, src_flat, dst_flat)


def kernel(h_inputs, objectives, edge_index, W1, al1, ar1, b1, W2, al2, ar2, b2):
    alr1 = jnp.stack([al1.reshape(-1), ar1.reshape(-1)], axis=1)
    alr2 = jnp.stack([al2.reshape(-1), ar2.reshape(-1)], axis=1)
    src_flat = edge_index[0]
    dst_flat = edge_index[1]

    feat1, ee1 = _tc_first(h_inputs, objectives, W1, alr1)
    part1, den1 = _sc_edge(feat1, ee1[:, 0], ee1[:, 1], src_flat, dst_flat)
    feat2, ee2 = _tc_mid(part1, den1.reshape(NT, N), b1.reshape(1, D), W2, alr2)
    part2, den2 = _sc_edge(feat2, ee2[:, 0], ee2[:, 1], src_flat, dst_flat)
    return _tc_final(part2, den2.reshape(NT, N), b2.reshape(1, D))
